# baseline (speedup 1.0000x reference)
import functools

import jax
import jax.numpy as jnp
from jax import lax
from jax.experimental import pallas as pl
from jax.experimental.pallas import tpu as pltpu
from jax.experimental.pallas import tpu_sc as plsc

_N = 4096
_BC = 512
_NBLK = _N // _BC
_K0 = _N // 3
_K1 = 2 * (_N // 3)
_MASK31 = 0x7FFFFFFF
_MIN32 = -(2 ** 31)


def _lsrv(x, n):
    return lax.shift_right_logical(x, jnp.full(x.shape, n, jnp.int32))


def _lsrs(x, n):
    return lax.shift_right_logical(x, jnp.int32(n))


def _sc_body(scores_hbm, out_hbm, s_v, ub_v, hist_v, out_v):
    on_worker0 = jnp.logical_and(
        lax.axis_index("s") == 0, lax.axis_index("c") == 0)

    @pl.when(on_worker0)
    def _():
        pltpu.sync_copy(scores_hbm, s_v)

        def init_body(i, carry):
            v = s_v[pl.ds(i * 16, 16)]
            b = lax.bitcast_convert_type(v, jnp.int32)
            key = b ^ (b >> 31) & jnp.int32(_MASK31)
            ub_v[pl.ds(i * 16, 16)] = key ^ jnp.int32(_MIN32)
            return carry

        lax.fori_loop(0, _N // 16, init_body, jnp.int32(0))

        iota = lax.iota(jnp.int32, 16)
        ones = jnp.ones((16,), jnp.int32)
        zeros = jnp.zeros((16,), jnp.int32)

        def select(kk):
            prefix = jnp.int32(0)
            base = jnp.int32(0)
            for shift in (24, 16, 8, 0):
                for g in range(16):
                    hist_v[pl.ds(g * 16, 16)] = zeros
                if shift == 24:
                    def hbody(i, carry):
                        v = ub_v[pl.ds(i * 16, 16)]
                        digit = _lsrv(v, 24) & jnp.int32(0xFF)
                        plsc.addupdate_scatter(hist_v, [digit], ones,
                                               mask=digit >= 0)
                        return carry
                else:
                    ph = _lsrs(prefix, shift + 8)

                    def hbody(i, carry, shift=shift, ph=ph):
                        v = ub_v[pl.ds(i * 16, 16)]
                        digit = _lsrv(v, shift) & jnp.int32(0xFF)
                        m = _lsrv(v, shift + 8) == ph
                        plsc.addupdate_scatter(hist_v, [digit], ones, mask=m)
                        return carry

                lax.fori_loop(0, _N // 16, hbody, jnp.int32(0))

                def gbody(g, carry):
                    cnt, found, dsel, bsel = carry
                    h = hist_v[pl.ds(g * 16, 16)]
                    csum = plsc.cumsum(h)
                    inc = csum + cnt
                    tot = jnp.sum(h)
                    found_here = jnp.logical_and(found == 0, cnt + tot > kk)
                    d_local = jnp.min(jnp.where(inc > kk, iota, jnp.int32(16)))
                    excl = jnp.sum(jnp.where(iota < d_local, h, jnp.int32(0)))
                    dsel = jnp.where(found_here, g * 16 + d_local, dsel)
                    bsel = jnp.where(found_here, cnt + excl, bsel)
                    found = jnp.where(found_here, jnp.int32(1), found)
                    return (cnt + tot, found, dsel, bsel)

                _, _, dsel, bsel = lax.fori_loop(
                    0, 16, gbody,
                    (base, jnp.int32(0), jnp.int32(0), jnp.int32(0)))
                prefix = prefix | lax.shift_left(dsel, jnp.int32(shift))
                base = bsel
            return prefix

        p0 = select(jnp.int32(_K0))
        p1 = select(jnp.int32(_K1))
        ub_vec = jnp.where(iota == 0, p0,
                           jnp.where(iota == 1, p1, jnp.int32(0)))
        key_vec = ub_vec ^ jnp.int32(_MIN32)
        val_vec = lax.bitcast_convert_type(
            jnp.where(key_vec >= 0, key_vec, key_vec ^ jnp.int32(_MASK31)),
            jnp.float32)
        out_v[...] = val_vec
        pltpu.sync_copy(out_v, out_hbm)


def _sc_thresholds(scores):
    mesh = plsc.VectorSubcoreMesh(core_axis_name="c", subcore_axis_name="s")
    return pl.kernel(
        _sc_body,
        mesh=mesh,
        out_type=jax.ShapeDtypeStruct((16,), jnp.float32),
        scratch_types=[
            pltpu.VMEM((_N,), jnp.float32),
            pltpu.VMEM((_N,), jnp.int32),
            pltpu.VMEM((256,), jnp.int32),
            pltpu.VMEM((16,), jnp.float32),
        ],
        compiler_params=pltpu.CompilerParams(needs_layout_passes=False),
    )(scores)


def _tc_body(thr_ref, scores_blk_ref, w_ref, o_ref):
    t0 = thr_ref[0]
    t1 = thr_ref[1]
    s = scores_blk_ref[...]
    half = jnp.where(s <= t0, 2.0, jnp.where(s <= t1, 8.0, 32.0))
    q_min = -half
    q_max = half - 1.0

    w = w_ref[...]
    mn = jnp.min(w, axis=0, keepdims=True)
    mx = jnp.max(w, axis=0, keepdims=True)
    scale = (mx - mn) / (q_max - q_min)
    scale = jnp.where(jnp.abs(scale) < 1e-6, jnp.float32(1e-6), scale)
    inv = 1.0 / scale
    zp = jnp.clip(jnp.round(q_min - mn / scale), q_min, q_max)
    q = jnp.clip(jnp.round(w * inv) + zp, -128.0, 127.0)
    o_ref[...] = (q - zp) * scale


def kernel(weight, scores):
    thr = _sc_thresholds(scores)
    scores_row = scores.reshape(1, _N)
    return pl.pallas_call(
        _tc_body,
        grid=(_NBLK,),
        in_specs=[
            pl.BlockSpec(memory_space=pltpu.SMEM),
            pl.BlockSpec((1, _BC), lambda j: (0, j)),
            pl.BlockSpec((_N, _BC), lambda j: (0, j)),
        ],
        out_specs=pl.BlockSpec((_N, _BC), lambda j: (0, j)),
        out_shape=jax.ShapeDtypeStruct((_N, _N), jnp.float32),
        compiler_params=pltpu.CompilerParams(
            dimension_semantics=("arbitrary",),
        ),
    )(thr, scores_row, weight)


# speedup vs baseline: 1.0046x; 1.0046x over previous
import functools

import jax
import jax.numpy as jnp
from jax import lax
from jax.experimental import pallas as pl
from jax.experimental.pallas import tpu as pltpu
from jax.experimental.pallas import tpu_sc as plsc

_N = 4096
_BC = 512
_NBLK = _N // _BC
_K0 = _N // 3
_K1 = 2 * (_N // 3)
_MASK31 = 0x7FFFFFFF
_MIN32 = -(2 ** 31)


def _lsrv(x, n):
    return lax.shift_right_logical(x, jnp.full(x.shape, n, jnp.int32))


def _lsrs(x, n):
    return lax.shift_right_logical(x, jnp.int32(n))


def _sc_body(scores_hbm, out_hbm, s_v, ub_v, hist_v, out_v):
    on_worker0 = jnp.logical_and(
        lax.axis_index("s") == 0, lax.axis_index("c") == 0)

    @pl.when(on_worker0)
    def _():
        pltpu.sync_copy(scores_hbm, s_v)

        def init_body(i, carry):
            v = s_v[pl.ds(i * 16, 16)]
            b = lax.bitcast_convert_type(v, jnp.int32)
            key = b ^ (b >> 31) & jnp.int32(_MASK31)
            ub_v[pl.ds(i * 16, 16)] = key ^ jnp.int32(_MIN32)
            return carry

        lax.fori_loop(0, _N // 16, init_body, jnp.int32(0), unroll=8)

        iota = lax.iota(jnp.int32, 16)
        ones = jnp.ones((16,), jnp.int32)
        zeros = jnp.zeros((16,), jnp.int32)

        def select(kk):
            prefix = jnp.int32(0)
            base = jnp.int32(0)
            for shift in (24, 16, 8, 0):
                for g in range(16):
                    hist_v[pl.ds(g * 16, 16)] = zeros
                if shift == 24:
                    def hbody(i, carry):
                        v = ub_v[pl.ds(i * 16, 16)]
                        digit = _lsrv(v, 24) & jnp.int32(0xFF)
                        plsc.addupdate_scatter(hist_v, [digit], ones,
                                               mask=digit >= 0)
                        return carry
                else:
                    ph = _lsrs(prefix, shift + 8)

                    def hbody(i, carry, shift=shift, ph=ph):
                        v = ub_v[pl.ds(i * 16, 16)]
                        digit = _lsrv(v, shift) & jnp.int32(0xFF)
                        m = _lsrv(v, shift + 8) == ph
                        plsc.addupdate_scatter(hist_v, [digit], ones, mask=m)
                        return carry

                lax.fori_loop(0, _N // 16, hbody, jnp.int32(0), unroll=8)

                def gbody(g, carry):
                    cnt, found, dsel, bsel = carry
                    h = hist_v[pl.ds(g * 16, 16)]
                    csum = plsc.cumsum(h)
                    inc = csum + cnt
                    tot = jnp.sum(h)
                    found_here = jnp.logical_and(found == 0, cnt + tot > kk)
                    d_local = jnp.min(jnp.where(inc > kk, iota, jnp.int32(16)))
                    excl = jnp.sum(jnp.where(iota < d_local, h, jnp.int32(0)))
                    dsel = jnp.where(found_here, g * 16 + d_local, dsel)
                    bsel = jnp.where(found_here, cnt + excl, bsel)
                    found = jnp.where(found_here, jnp.int32(1), found)
                    return (cnt + tot, found, dsel, bsel)

                _, _, dsel, bsel = lax.fori_loop(
                    0, 16, gbody,
                    (base, jnp.int32(0), jnp.int32(0), jnp.int32(0)))
                prefix = prefix | lax.shift_left(dsel, jnp.int32(shift))
                base = bsel
            return prefix

        p0 = select(jnp.int32(_K0))
        p1 = select(jnp.int32(_K1))
        ub_vec = jnp.where(iota == 0, p0,
                           jnp.where(iota == 1, p1, jnp.int32(0)))
        key_vec = ub_vec ^ jnp.int32(_MIN32)
        val_vec = lax.bitcast_convert_type(
            jnp.where(key_vec >= 0, key_vec, key_vec ^ jnp.int32(_MASK31)),
            jnp.float32)
        out_v[...] = val_vec
        pltpu.sync_copy(out_v, out_hbm)


def _sc_thresholds(scores):
    mesh = plsc.VectorSubcoreMesh(core_axis_name="c", subcore_axis_name="s")
    return pl.kernel(
        _sc_body,
        mesh=mesh,
        out_type=jax.ShapeDtypeStruct((16,), jnp.float32),
        scratch_types=[
            pltpu.VMEM((_N,), jnp.float32),
            pltpu.VMEM((_N,), jnp.int32),
            pltpu.VMEM((256,), jnp.int32),
            pltpu.VMEM((16,), jnp.float32),
        ],
        compiler_params=pltpu.CompilerParams(needs_layout_passes=False),
    )(scores)


def _tc_body(thr_ref, scores_blk_ref, w_ref, o_ref):
    t0 = thr_ref[0]
    t1 = thr_ref[1]
    s = scores_blk_ref[...]
    half = jnp.where(s <= t0, 2.0, jnp.where(s <= t1, 8.0, 32.0))
    q_min = -half
    q_max = half - 1.0

    w = w_ref[...]
    mn = jnp.min(w, axis=0, keepdims=True)
    mx = jnp.max(w, axis=0, keepdims=True)
    scale = (mx - mn) / (q_max - q_min)
    scale = jnp.where(jnp.abs(scale) < 1e-6, jnp.float32(1e-6), scale)
    inv = 1.0 / scale
    zp = jnp.clip(jnp.round(q_min - mn / scale), q_min, q_max)
    q = jnp.clip(jnp.round(w * inv) + zp, -128.0, 127.0)
    o_ref[...] = (q - zp) * scale


def kernel(weight, scores):
    thr = _sc_thresholds(scores)
    scores_row = scores.reshape(1, _N)
    return pl.pallas_call(
        _tc_body,
        grid=(_NBLK,),
        in_specs=[
            pl.BlockSpec(memory_space=pltpu.SMEM),
            pl.BlockSpec((1, _BC), lambda j: (0, j)),
            pl.BlockSpec((_N, _BC), lambda j: (0, j)),
        ],
        out_specs=pl.BlockSpec((_N, _BC), lambda j: (0, j)),
        out_shape=jax.ShapeDtypeStruct((_N, _N), jnp.float32),
        compiler_params=pltpu.CompilerParams(
            dimension_semantics=("arbitrary",),
        ),
    )(thr, scores_row, weight)


# speedup vs baseline: 1.7595x; 1.7514x over previous
import jax
import jax.numpy as jnp
from jax.experimental import pallas as pl
from jax.experimental.pallas import tpu as pltpu

_N = 4096
_BC = 512
_NBLK = _N // _BC
_RC = 1024
_NCH = _N // _RC
_K0 = _N // 3
_K1 = 2 * (_N // 3)
_MASK = 0x7FFFFFFF


def _kth_key(keys, k):
    n_neg = jnp.sum((keys < jnp.int32(0)).astype(jnp.int32))
    is_neg = jnp.int32(k + 1) <= n_neg
    lo0 = jnp.where(is_neg, jnp.int32(-(2 ** 31)), jnp.int32(0))
    hi0 = jnp.where(is_neg, jnp.int32(-1), jnp.int32(2 ** 31 - 1))

    def body(_, lohi):
        lo, hi = lohi
        mid = lo + (hi - lo) // 2
        cnt = jnp.sum((keys <= mid).astype(jnp.int32))
        ge = cnt >= jnp.int32(k + 1)
        return jnp.where(ge, lo, mid + 1), jnp.where(ge, mid, hi)

    lo, _ = jax.lax.fori_loop(0, 31, body, (lo0, hi0))
    return lo


def _load(w_ref, in_buf, in_sems, j, b):
    return pltpu.make_async_copy(
        w_ref.at[:, pl.ds(j * _BC, _BC)], in_buf.at[b], in_sems.at[b])


def _store(o_ref, out_buf, out_sems, j, b, r):
    return pltpu.make_async_copy(
        out_buf.at[b, pl.ds(r * _RC, _RC), :],
        o_ref.at[pl.ds(r * _RC, _RC), pl.ds(j * _BC, _BC)],
        out_sems.at[b, r])


def _body(scores8_ref, scores_full_ref, w_ref, o_ref,
          in_buf, out_buf, in_sems, out_sems):
    _load(w_ref, in_buf, in_sems, 0, 0).start()
    _load(w_ref, in_buf, in_sems, 1, 1).start()

    sf = scores_full_ref[...]
    bbits = jax.lax.bitcast_convert_type(sf, jnp.int32)
    keys = bbits ^ ((bbits >> 31) & jnp.int32(_MASK))

    def unmap(kk):
        return jax.lax.bitcast_convert_type(
            jnp.where(kk >= 0, kk, kk ^ jnp.int32(_MASK)), jnp.float32)

    t0 = unmap(_kth_key(keys, _K0))
    t1 = unmap(_kth_key(keys, _K1))

    def block(j, _):
        b = j & 1
        _load(w_ref, in_buf, in_sems, j, b).wait()

        s = scores8_ref[pl.ds(j, 1), :]
        half = jnp.where(s <= t0, 2.0, jnp.where(s <= t1, 8.0, 32.0))
        q_min = -half
        q_max = half - 1.0

        w = in_buf[b]
        mn = jnp.min(w, axis=0, keepdims=True)
        mx = jnp.max(w, axis=0, keepdims=True)
        scale = (mx - mn) / (q_max - q_min)
        scale = jnp.where(jnp.abs(scale) < 1e-6, jnp.float32(1e-6), scale)
        inv = 1.0 / scale
        zp = jnp.clip(jnp.round(q_min - mn / scale), q_min, q_max)

        @pl.when(j >= 2)
        def _():
            for r in range(_NCH):
                _store(o_ref, out_buf, out_sems, j - 2, b, r).wait()

        for r in range(_NCH):
            wc = w[r * _RC:(r + 1) * _RC, :]
            q = jnp.clip(jnp.round(wc * inv) + zp, -128.0, 127.0)
            out_buf[b, r * _RC:(r + 1) * _RC, :] = (q - zp) * scale
            _store(o_ref, out_buf, out_sems, j, b, r).start()

        @pl.when(j + 2 < _NBLK)
        def _():
            _load(w_ref, in_buf, in_sems, j + 2, b).start()

        return 0

    jax.lax.fori_loop(0, _NBLK, block, 0)
    for jj in (_NBLK - 2, _NBLK - 1):
        for r in range(_NCH):
            _store(o_ref, out_buf, out_sems, jj, jj & 1, r).wait()


def kernel(weight, scores):
    scores8 = scores.reshape(_NBLK, _BC)
    scores_full = scores.reshape(32, 128)
    return pl.pallas_call(
        _body,
        in_specs=[
            pl.BlockSpec(memory_space=pltpu.VMEM),
            pl.BlockSpec(memory_space=pltpu.VMEM),
            pl.BlockSpec(memory_space=pl.ANY),
        ],
        out_specs=pl.BlockSpec(memory_space=pl.ANY),
        out_shape=jax.ShapeDtypeStruct((_N, _N), jnp.float32),
        scratch_shapes=[
            pltpu.VMEM((2, _N, _BC), jnp.float32),
            pltpu.VMEM((2, _N, _BC), jnp.float32),
            pltpu.SemaphoreType.DMA((2,)),
            pltpu.SemaphoreType.DMA((2, _NCH)),
        ],
    )(scores8, scores_full, weight)


# speedup vs baseline: 1.7615x; 1.0011x over previous
import jax
import jax.numpy as jnp
from jax.experimental import pallas as pl
from jax.experimental.pallas import tpu as pltpu

_N = 4096
_BC = 512
_NBLK = _N // _BC
_RC = 1024
_NCH = _N // _RC
_LRC = 2048
_NLC = _N // _LRC
_K0 = _N // 3
_K1 = 2 * (_N // 3)
_MASK = 0x7FFFFFFF


def _kth_key(keys, k):
    n_neg = jnp.sum((keys < jnp.int32(0)).astype(jnp.int32))
    is_neg = jnp.int32(k + 1) <= n_neg
    lo0 = jnp.where(is_neg, jnp.int32(-(2 ** 31)), jnp.int32(0))
    hi0 = jnp.where(is_neg, jnp.int32(-1), jnp.int32(2 ** 31 - 1))

    def body(_, lohi):
        lo, hi = lohi
        mid = lo + (hi - lo) // 2
        cnt = jnp.sum((keys <= mid).astype(jnp.int32))
        ge = cnt >= jnp.int32(k + 1)
        return jnp.where(ge, lo, mid + 1), jnp.where(ge, mid, hi)

    lo, _ = jax.lax.fori_loop(0, 31, body, (lo0, hi0))
    return lo


def _load(w_ref, in_buf, in_sems, j, b, c):
    return pltpu.make_async_copy(
        w_ref.at[pl.ds(c * _LRC, _LRC), pl.ds(j * _BC, _BC)],
        in_buf.at[b, pl.ds(c * _LRC, _LRC), :],
        in_sems.at[b, c])


def _start_load(w_ref, in_buf, in_sems, j, b):
    for c in range(_NLC):
        _load(w_ref, in_buf, in_sems, j, b, c).start()


def _store(o_ref, out_buf, out_sems, j, b, r):
    return pltpu.make_async_copy(
        out_buf.at[b, pl.ds(r * _RC, _RC), :],
        o_ref.at[pl.ds(r * _RC, _RC), pl.ds(j * _BC, _BC)],
        out_sems.at[b, r])


def _body(scores8_ref, scores_full_ref, w_ref, o_ref,
          in_buf, out_buf, in_sems, out_sems):
    _start_load(w_ref, in_buf, in_sems, 0, 0)
    _start_load(w_ref, in_buf, in_sems, 1, 1)

    sf = scores_full_ref[...]
    bbits = jax.lax.bitcast_convert_type(sf, jnp.int32)
    keys = bbits ^ ((bbits >> 31) & jnp.int32(_MASK))

    def unmap(kk):
        return jax.lax.bitcast_convert_type(
            jnp.where(kk >= 0, kk, kk ^ jnp.int32(_MASK)), jnp.float32)

    t0 = unmap(_kth_key(keys, _K0))
    t1 = unmap(_kth_key(keys, _K1))

    def block(j, _):
        b = j & 1
        _load(w_ref, in_buf, in_sems, j, b, 0).wait()
        w0 = in_buf[b, 0:_LRC, :]
        mn = jnp.min(w0, axis=0, keepdims=True)
        mx = jnp.max(w0, axis=0, keepdims=True)
        _load(w_ref, in_buf, in_sems, j, b, 1).wait()
        w1 = in_buf[b, _LRC:_N, :]
        mn = jnp.minimum(mn, jnp.min(w1, axis=0, keepdims=True))
        mx = jnp.maximum(mx, jnp.max(w1, axis=0, keepdims=True))

        s = scores8_ref[pl.ds(j, 1), :]
        half = jnp.where(s <= t0, 2.0, jnp.where(s <= t1, 8.0, 32.0))
        q_min = -half
        q_max = half - 1.0
        scale = (mx - mn) / (q_max - q_min)
        scale = jnp.where(jnp.abs(scale) < 1e-6, jnp.float32(1e-6), scale)
        inv = 1.0 / scale
        zp = jnp.clip(jnp.round(q_min - mn / scale), q_min, q_max)

        @pl.when(j >= 2)
        def _():
            for r in range(_NCH):
                _store(o_ref, out_buf, out_sems, j - 2, b, r).wait()

        w = in_buf[b]
        for r in range(_NCH):
            wc = w[r * _RC:(r + 1) * _RC, :]
            q = jnp.clip(jnp.round(wc * inv) + zp, -128.0, 127.0)
            out_buf[b, r * _RC:(r + 1) * _RC, :] = (q - zp) * scale
            _store(o_ref, out_buf, out_sems, j, b, r).start()

        @pl.when(j + 2 < _NBLK)
        def _():
            _start_load(w_ref, in_buf, in_sems, j + 2, b)

        return 0

    jax.lax.fori_loop(0, _NBLK, block, 0)
    for jj in (_NBLK - 2, _NBLK - 1):
        for r in range(_NCH):
            _store(o_ref, out_buf, out_sems, jj, jj & 1, r).wait()


def kernel(weight, scores):
    scores8 = scores.reshape(_NBLK, _BC)
    scores_full = scores.reshape(32, 128)
    return pl.pallas_call(
        _body,
        in_specs=[
            pl.BlockSpec(memory_space=pltpu.VMEM),
            pl.BlockSpec(memory_space=pltpu.VMEM),
            pl.BlockSpec(memory_space=pl.ANY),
        ],
        out_specs=pl.BlockSpec(memory_space=pl.ANY),
        out_shape=jax.ShapeDtypeStruct((_N, _N), jnp.float32),
        scratch_shapes=[
            pltpu.VMEM((2, _N, _BC), jnp.float32),
            pltpu.VMEM((2, _N, _BC), jnp.float32),
            pltpu.SemaphoreType.DMA((2, _NLC)),
            pltpu.SemaphoreType.DMA((2, _NCH)),
        ],
    )(scores8, scores_full, weight)


# speedup vs baseline: 1.9190x; 1.0894x over previous
import jax
import jax.numpy as jnp
from jax.experimental import pallas as pl
from jax.experimental.pallas import tpu as pltpu

_N = 4096
_BC = 512
_NBLK = _N // _BC
_RC = 1024
_NCH = _N // _RC
_LRC = 2048
_NLC = _N // _LRC
_K0 = _N // 3
_K1 = 2 * (_N // 3)
_MASK = 0x7FFFFFFF


def _kth_key(keys, k):
    n_neg = jnp.sum((keys < jnp.int32(0)).astype(jnp.int32))
    is_neg = jnp.int32(k + 1) <= n_neg
    lo0 = jnp.where(is_neg, jnp.int32(-(2 ** 31)), jnp.int32(0))
    hi0 = jnp.where(is_neg, jnp.int32(-1), jnp.int32(2 ** 31 - 1))

    def body(_, lohi):
        lo, hi = lohi
        mid = lo + (hi - lo) // 2
        cnt = jnp.sum((keys <= mid).astype(jnp.int32))
        ge = cnt >= jnp.int32(k + 1)
        return jnp.where(ge, lo, mid + 1), jnp.where(ge, mid, hi)

    lo, _ = jax.lax.fori_loop(0, 31, body, (lo0, hi0))
    return lo


def _load(w_ref, in_buf, in_sems, j, b, c):
    return pltpu.make_async_copy(
        w_ref.at[pl.ds(c * _LRC, _LRC), pl.ds(j * _BC, _BC)],
        in_buf.at[b, pl.ds(c * _LRC, _LRC), :],
        in_sems.at[b, c])


def _start_load(w_ref, in_buf, in_sems, j, b):
    for c in range(_NLC):
        _load(w_ref, in_buf, in_sems, j, b, c).start()


def _store(o_ref, out_buf, out_sems, j, b, r):
    return pltpu.make_async_copy(
        out_buf.at[b, pl.ds(r * _RC, _RC), :],
        o_ref.at[pl.ds(r * _RC, _RC), pl.ds(j * _BC, _BC)],
        out_sems.at[b, r])


def _body(scores8_ref, scores_full_ref, w_ref, o_ref,
          in_buf, out_buf, in_sems, out_sems):
    _start_load(w_ref, in_buf, in_sems, 0, 0)
    _start_load(w_ref, in_buf, in_sems, 1, 1)
    _start_load(w_ref, in_buf, in_sems, 2, 2)

    sf = scores_full_ref[...]
    bbits = jax.lax.bitcast_convert_type(sf, jnp.int32)
    keys = bbits ^ ((bbits >> 31) & jnp.int32(_MASK))

    def unmap(kk):
        return jax.lax.bitcast_convert_type(
            jnp.where(kk >= 0, kk, kk ^ jnp.int32(_MASK)), jnp.float32)

    t0 = unmap(_kth_key(keys, _K0))
    t1 = unmap(_kth_key(keys, _K1))

    def block(j, _):
        b = j % 3
        _load(w_ref, in_buf, in_sems, j, b, 0).wait()
        w0 = in_buf[b, 0:_LRC, :]
        mn = jnp.min(w0, axis=0, keepdims=True)
        mx = jnp.max(w0, axis=0, keepdims=True)
        _load(w_ref, in_buf, in_sems, j, b, 1).wait()
        w1 = in_buf[b, _LRC:_N, :]
        mn = jnp.minimum(mn, jnp.min(w1, axis=0, keepdims=True))
        mx = jnp.maximum(mx, jnp.max(w1, axis=0, keepdims=True))

        s = scores8_ref[pl.ds(j, 1), :]
        half = jnp.where(s <= t0, 2.0, jnp.where(s <= t1, 8.0, 32.0))
        q_min = -half
        q_max = half - 1.0
        scale = (mx - mn) / (q_max - q_min)
        scale = jnp.where(jnp.abs(scale) < 1e-6, jnp.float32(1e-6), scale)
        inv = 1.0 / scale
        zp = jnp.clip(jnp.round(q_min - mn / scale), q_min, q_max)

        b2 = j & 1
        @pl.when(j >= 2)
        def _():
            for r in range(_NCH):
                _store(o_ref, out_buf, out_sems, j - 2, b2, r).wait()

        w = in_buf[b]
        for r in range(_NCH):
            wc = w[r * _RC:(r + 1) * _RC, :]
            q = jnp.clip(jnp.round(wc * inv) + zp, -128.0, 127.0)
            out_buf[b2, r * _RC:(r + 1) * _RC, :] = (q - zp) * scale
            _store(o_ref, out_buf, out_sems, j, b2, r).start()

        @pl.when(j + 3 < _NBLK)
        def _():
            _start_load(w_ref, in_buf, in_sems, j + 3, b)

        return 0

    jax.lax.fori_loop(0, _NBLK, block, 0)
    for jj in (_NBLK - 2, _NBLK - 1):
        for r in range(_NCH):
            _store(o_ref, out_buf, out_sems, jj, jj & 1, r).wait()


def kernel(weight, scores):
    scores8 = scores.reshape(_NBLK, _BC)
    scores_full = scores.reshape(32, 128)
    return pl.pallas_call(
        _body,
        in_specs=[
            pl.BlockSpec(memory_space=pltpu.VMEM),
            pl.BlockSpec(memory_space=pltpu.VMEM),
            pl.BlockSpec(memory_space=pl.ANY),
        ],
        out_specs=pl.BlockSpec(memory_space=pl.ANY),
        out_shape=jax.ShapeDtypeStruct((_N, _N), jnp.float32),
        scratch_shapes=[
            pltpu.VMEM((3, _N, _BC), jnp.float32),
            pltpu.VMEM((2, _N, _BC), jnp.float32),
            pltpu.SemaphoreType.DMA((3, _NLC)),
            pltpu.SemaphoreType.DMA((2, _NCH)),
        ],
    )(scores8, scores_full, weight)


# speedup vs baseline: 2.0428x; 1.0645x over previous
import jax
import jax.numpy as jnp
from jax.experimental import pallas as pl
from jax.experimental.pallas import tpu as pltpu

_N = 4096
_BC = 512
_NBLK = _N // _BC
_RC = 1024
_NCH = _N // _RC
_LRC = 2048
_NLC = _N // _LRC
_NIN = 4
_NOUT = 2
_K0 = _N // 3
_K1 = 2 * (_N // 3)
_MASK = 0x7FFFFFFF


def _kth_key(keys, k):
    n_neg = jnp.sum((keys < jnp.int32(0)).astype(jnp.int32))
    is_neg = jnp.int32(k + 1) <= n_neg
    lo0 = jnp.where(is_neg, jnp.int32(-(2 ** 31)), jnp.int32(0))
    hi0 = jnp.where(is_neg, jnp.int32(-1), jnp.int32(2 ** 31 - 1))

    def body(_, lohi):
        lo, hi = lohi
        mid = lo + (hi - lo) // 2
        cnt = jnp.sum((keys <= mid).astype(jnp.int32))
        ge = cnt >= jnp.int32(k + 1)
        return jnp.where(ge, lo, mid + 1), jnp.where(ge, mid, hi)

    lo, _ = jax.lax.fori_loop(0, 31, body, (lo0, hi0))
    return lo


def _load(w_ref, in_buf, in_sems, j, b, c):
    return pltpu.make_async_copy(
        w_ref.at[pl.ds(c * _LRC, _LRC), pl.ds(j * _BC, _BC)],
        in_buf.at[b, pl.ds(c * _LRC, _LRC), :],
        in_sems.at[b, c])


def _start_load(w_ref, in_buf, in_sems, j, b):
    for c in range(_NLC):
        _load(w_ref, in_buf, in_sems, j, b, c).start()


def _store(o_ref, out_buf, out_sems, j, b, r):
    return pltpu.make_async_copy(
        out_buf.at[b, pl.ds(r * _RC, _RC), :],
        o_ref.at[pl.ds(r * _RC, _RC), pl.ds(j * _BC, _BC)],
        out_sems.at[b, r])


def _body(scores8_ref, scores_full_ref, w_ref, o_ref,
          in_buf, out_buf, in_sems, out_sems):
    for jj in range(min(_NIN, _NBLK)):
        _start_load(w_ref, in_buf, in_sems, jj, jj)

    sf = scores_full_ref[...]
    bbits = jax.lax.bitcast_convert_type(sf, jnp.int32)
    keys = bbits ^ ((bbits >> 31) & jnp.int32(_MASK))

    def unmap(kk):
        return jax.lax.bitcast_convert_type(
            jnp.where(kk >= 0, kk, kk ^ jnp.int32(_MASK)), jnp.float32)

    t0 = unmap(_kth_key(keys, _K0))
    t1 = unmap(_kth_key(keys, _K1))

    def block(j, _):
        b = j % _NIN
        b2 = j % _NOUT
        _load(w_ref, in_buf, in_sems, j, b, 0).wait()
        w0 = in_buf[b, 0:_LRC, :]
        mn = jnp.min(w0, axis=0, keepdims=True)
        mx = jnp.max(w0, axis=0, keepdims=True)
        _load(w_ref, in_buf, in_sems, j, b, 1).wait()
        w1 = in_buf[b, _LRC:_N, :]
        mn = jnp.minimum(mn, jnp.min(w1, axis=0, keepdims=True))
        mx = jnp.maximum(mx, jnp.max(w1, axis=0, keepdims=True))

        s = scores8_ref[pl.ds(j, 1), :]
        half = jnp.where(s <= t0, 2.0, jnp.where(s <= t1, 8.0, 32.0))
        q_min = -half
        q_max = half - 1.0
        scale = (mx - mn) / (q_max - q_min)
        scale = jnp.where(jnp.abs(scale) < 1e-6, jnp.float32(1e-6), scale)
        inv = 1.0 / scale
        zp = jnp.clip(jnp.round(q_min - mn / scale), q_min, q_max)

        @pl.when(j >= _NOUT)
        def _():
            for r in range(_NCH):
                _store(o_ref, out_buf, out_sems, j - _NOUT, b2, r).wait()

        w = in_buf[b]
        for r in range(_NCH):
            wc = w[r * _RC:(r + 1) * _RC, :]
            q = jnp.clip(jnp.round(wc * inv) + zp, -128.0, 127.0)
            out_buf[b2, r * _RC:(r + 1) * _RC, :] = (q - zp) * scale
            _store(o_ref, out_buf, out_sems, j, b2, r).start()

        @pl.when(j + _NIN < _NBLK)
        def _():
            _start_load(w_ref, in_buf, in_sems, j + _NIN, b)

        return 0

    jax.lax.fori_loop(0, _NBLK, block, 0)
    for jj in range(max(0, _NBLK - _NOUT), _NBLK):
        for r in range(_NCH):
            _store(o_ref, out_buf, out_sems, jj, jj % _NOUT, r).wait()


def kernel(weight, scores):
    scores8 = scores.reshape(_NBLK, _BC)
    scores_full = scores.reshape(32, 128)
    return pl.pallas_call(
        _body,
        in_specs=[
            pl.BlockSpec(memory_space=pltpu.VMEM),
            pl.BlockSpec(memory_space=pltpu.VMEM),
            pl.BlockSpec(memory_space=pl.ANY),
        ],
        out_specs=pl.BlockSpec(memory_space=pl.ANY),
        out_shape=jax.ShapeDtypeStruct((_N, _N), jnp.float32),
        scratch_shapes=[
            pltpu.VMEM((_NIN, _N, _BC), jnp.float32),
            pltpu.VMEM((_NOUT, _N, _BC), jnp.float32),
            pltpu.SemaphoreType.DMA((_NIN, _NLC)),
            pltpu.SemaphoreType.DMA((_NOUT, _NCH)),
        ],
    )(scores8, scores_full, weight)


# speedup vs baseline: 2.0609x; 1.0089x over previous
import jax
import jax.numpy as jnp
from jax.experimental import pallas as pl
from jax.experimental.pallas import tpu as pltpu

_N = 4096
_BC = 512
_NBLK = _N // _BC
_RC = 1024
_NCH = _N // _RC
_LRC = 2048
_NLC = _N // _LRC
_NIN = 5
_NOUT = 2
_K0 = _N // 3
_K1 = 2 * (_N // 3)
_MASK = 0x7FFFFFFF


def _kth_key(keys, k):
    n_neg = jnp.sum((keys < jnp.int32(0)).astype(jnp.int32))
    is_neg = jnp.int32(k + 1) <= n_neg
    lo0 = jnp.where(is_neg, jnp.int32(-(2 ** 31)), jnp.int32(0))
    hi0 = jnp.where(is_neg, jnp.int32(-1), jnp.int32(2 ** 31 - 1))

    def body(_, lohi):
        lo, hi = lohi
        mid = lo + (hi - lo) // 2
        cnt = jnp.sum((keys <= mid).astype(jnp.int32))
        ge = cnt >= jnp.int32(k + 1)
        return jnp.where(ge, lo, mid + 1), jnp.where(ge, mid, hi)

    lo, _ = jax.lax.fori_loop(0, 31, body, (lo0, hi0))
    return lo


def _load(w_ref, in_buf, in_sems, j, b, c):
    return pltpu.make_async_copy(
        w_ref.at[pl.ds(c * _LRC, _LRC), pl.ds(j * _BC, _BC)],
        in_buf.at[b, pl.ds(c * _LRC, _LRC), :],
        in_sems.at[b, c])


def _start_load(w_ref, in_buf, in_sems, j, b):
    for c in range(_NLC):
        _load(w_ref, in_buf, in_sems, j, b, c).start()


def _store(o_ref, out_buf, out_sems, j, b, r):
    return pltpu.make_async_copy(
        out_buf.at[b, pl.ds(r * _RC, _RC), :],
        o_ref.at[pl.ds(r * _RC, _RC), pl.ds(j * _BC, _BC)],
        out_sems.at[b, r])


def _body(scores8_ref, scores_full_ref, w_ref, o_ref,
          in_buf, out_buf, in_sems, out_sems):
    for jj in range(min(_NIN, _NBLK)):
        _start_load(w_ref, in_buf, in_sems, jj, jj)

    sf = scores_full_ref[...]
    bbits = jax.lax.bitcast_convert_type(sf, jnp.int32)
    keys = bbits ^ ((bbits >> 31) & jnp.int32(_MASK))

    def unmap(kk):
        return jax.lax.bitcast_convert_type(
            jnp.where(kk >= 0, kk, kk ^ jnp.int32(_MASK)), jnp.float32)

    t0 = unmap(_kth_key(keys, _K0))
    t1 = unmap(_kth_key(keys, _K1))

    def block(j, _):
        b = j % _NIN
        b2 = j % _NOUT
        _load(w_ref, in_buf, in_sems, j, b, 0).wait()
        w0 = in_buf[b, 0:_LRC, :]
        mn = jnp.min(w0, axis=0, keepdims=True)
        mx = jnp.max(w0, axis=0, keepdims=True)
        _load(w_ref, in_buf, in_sems, j, b, 1).wait()
        w1 = in_buf[b, _LRC:_N, :]
        mn = jnp.minimum(mn, jnp.min(w1, axis=0, keepdims=True))
        mx = jnp.maximum(mx, jnp.max(w1, axis=0, keepdims=True))

        s = scores8_ref[pl.ds(j, 1), :]
        half = jnp.where(s <= t0, 2.0, jnp.where(s <= t1, 8.0, 32.0))
        q_min = -half
        q_max = half - 1.0
        scale = (mx - mn) / (q_max - q_min)
        scale = jnp.where(jnp.abs(scale) < 1e-6, jnp.float32(1e-6), scale)
        inv = 1.0 / scale
        zp = jnp.clip(jnp.round(q_min - mn / scale), q_min, q_max)

        @pl.when(j >= _NOUT)
        def _():
            for r in range(_NCH):
                _store(o_ref, out_buf, out_sems, j - _NOUT, b2, r).wait()

        w = in_buf[b]
        for r in range(_NCH):
            wc = w[r * _RC:(r + 1) * _RC, :]
            q = jnp.clip(jnp.round(wc * inv) + zp, -128.0, 127.0)
            out_buf[b2, r * _RC:(r + 1) * _RC, :] = (q - zp) * scale
            _store(o_ref, out_buf, out_sems, j, b2, r).start()

        @pl.when(j + _NIN < _NBLK)
        def _():
            _start_load(w_ref, in_buf, in_sems, j + _NIN, b)

        return 0

    jax.lax.fori_loop(0, _NBLK, block, 0)
    for jj in range(max(0, _NBLK - _NOUT), _NBLK):
        for r in range(_NCH):
            _store(o_ref, out_buf, out_sems, jj, jj % _NOUT, r).wait()


def kernel(weight, scores):
    scores8 = scores.reshape(_NBLK, _BC)
    scores_full = scores.reshape(32, 128)
    return pl.pallas_call(
        _body,
        in_specs=[
            pl.BlockSpec(memory_space=pltpu.VMEM),
            pl.BlockSpec(memory_space=pltpu.VMEM),
            pl.BlockSpec(memory_space=pl.ANY),
        ],
        out_specs=pl.BlockSpec(memory_space=pl.ANY),
        out_shape=jax.ShapeDtypeStruct((_N, _N), jnp.float32),
        scratch_shapes=[
            pltpu.VMEM((_NIN, _N, _BC), jnp.float32),
            pltpu.VMEM((_NOUT, _N, _BC), jnp.float32),
            pltpu.SemaphoreType.DMA((_NIN, _NLC)),
            pltpu.SemaphoreType.DMA((_NOUT, _NCH)),
        ],
    )(scores8, scores_full, weight)
